# Initial kernel scaffold; baseline (speedup 1.0000x reference)
import jax
import jax.numpy as jnp
from jax.experimental import pallas as pl


def kernel(x, edge_index, edge_attr, batch, W1, a_src1, a_dst1, b1, W2, a_src2, a_dst2, b2):
    raise NotImplementedError("write your pallas kernel here")


# speedup vs baseline: 54.8068x; 54.8068x over previous
import functools

import jax
import jax.numpy as jnp
from jax import lax
from jax.experimental import pallas as pl
from jax.experimental.pallas import tpu as pltpu
from jax.experimental.pallas import tpu_sc as plsc

N = 10000
E = 320000
NUM_GRAPHS = 64
IN_CH = 128
HID = 16
HEADS = 8

NPAD = 10240
NW = 32
CH = 128
NCHUNK = 81
EPAD = NW * CH * NCHUNK
BLK = 1024
GRID = NPAD // BLK


def _mm2_body(x_ref, a_ref, b_ref, o1_ref, o2_ref):
    x = x_ref[...]
    o1_ref[...] = jnp.dot(x, a_ref[...], preferred_element_type=jnp.float32)
    o2_ref[...] = jnp.dot(x, b_ref[...], preferred_element_type=jnp.float32)


def _mm2(x, a, b):
    m = x.shape[0]
    return pl.pallas_call(
        _mm2_body,
        grid=(m // BLK,),
        in_specs=[
            pl.BlockSpec((BLK, x.shape[1]), lambda i: (i, 0)),
            pl.BlockSpec(a.shape, lambda i: (0, 0)),
            pl.BlockSpec(b.shape, lambda i: (0, 0)),
        ],
        out_specs=[
            pl.BlockSpec((BLK, a.shape[1]), lambda i: (i, 0)),
            pl.BlockSpec((BLK, b.shape[1]), lambda i: (i, 0)),
        ],
        out_shape=[
            jax.ShapeDtypeStruct((m, a.shape[1]), jnp.float32),
            jax.ShapeDtypeStruct((m, b.shape[1]), jnp.float32),
        ],
    )(x, a, b)


def _norm_mm2_body(p_ref, e_ref, bias_ref, a_ref, b_ref, o1_ref, o2_ref):
    acc = p_ref[0] + p_ref[1]
    msg = acc[:, :IN_CH]
    den = acc[:, IN_CH:IN_CH + HEADS]
    den_exp = jnp.dot(den, e_ref[...], preferred_element_type=jnp.float32)
    x2 = jnp.maximum(msg / (den_exp + 1e-16) + bias_ref[...], 0.0)
    o1_ref[...] = jnp.dot(x2, a_ref[...], preferred_element_type=jnp.float32)
    o2_ref[...] = jnp.dot(x2, b_ref[...], preferred_element_type=jnp.float32)


def _norm_mm2(p, e, bias, a, b):
    return pl.pallas_call(
        _norm_mm2_body,
        grid=(GRID,),
        in_specs=[
            pl.BlockSpec((2, BLK, IN_CH + HID), lambda i: (0, i, 0)),
            pl.BlockSpec(e.shape, lambda i: (0, 0)),
            pl.BlockSpec(bias.shape, lambda i: (0, 0)),
            pl.BlockSpec(a.shape, lambda i: (0, 0)),
            pl.BlockSpec(b.shape, lambda i: (0, 0)),
        ],
        out_specs=[
            pl.BlockSpec((BLK, a.shape[1]), lambda i: (i, 0)),
            pl.BlockSpec((BLK, b.shape[1]), lambda i: (i, 0)),
        ],
        out_shape=[
            jax.ShapeDtypeStruct((NPAD, a.shape[1]), jnp.float32),
            jax.ShapeDtypeStruct((NPAD, b.shape[1]), jnp.float32),
        ],
    )(p, e, bias, a, b)


def _pool_body(p_ref, batch_ref, bias_ref, o_ref, sums, cnts):
    i = pl.program_id(0)
    acc = p_ref[0] + p_ref[1]
    o2 = acc[:, :HID] / (acc[:, HID:HID + 1] + 1e-16) + bias_ref[...]
    bv = batch_ref[0, 0, :]
    gid = lax.broadcasted_iota(jnp.int32, (BLK, NUM_GRAPHS), 1)
    oh = (bv[:, None] == gid).astype(jnp.float32)
    s = lax.dot_general(oh, o2, (((0,), (0,)), ((), ())),
                        preferred_element_type=jnp.float32)
    c = lax.dot_general(oh, jnp.ones((BLK, 1), jnp.float32),
                        (((0,), (0,)), ((), ())),
                        preferred_element_type=jnp.float32)

    @pl.when(i == 0)
    def _():
        sums[...] = s
        cnts[...] = c

    @pl.when(i > 0)
    def _():
        sums[...] += s
        cnts[...] += c

    @pl.when(i == GRID - 1)
    def _():
        o_ref[...] = sums[...] / jnp.maximum(cnts[...], 1.0)


def _pool(p, batch3, bias):
    return pl.pallas_call(
        _pool_body,
        grid=(GRID,),
        in_specs=[
            pl.BlockSpec((2, BLK, 2 * HID), lambda i: (0, i, 0)),
            pl.BlockSpec((1, 1, BLK), lambda i: (i, 0, 0)),
            pl.BlockSpec(bias.shape, lambda i: (0, 0)),
        ],
        out_specs=pl.BlockSpec((NUM_GRAPHS, HID), lambda i: (0, 0)),
        out_shape=jax.ShapeDtypeStruct((NUM_GRAPHS, HID), jnp.float32),
        scratch_shapes=[
            pltpu.VMEM((NUM_GRAPHS, HID), jnp.float32),
            pltpu.VMEM((NUM_GRAPHS, 1), jnp.float32),
        ],
    )(p, batch3, bias)


def _make_edge_pass(dh):
    R = dh + 16
    epw = EPAD // NW
    rows_per_tile = NPAD // 16
    nzcopy = rows_per_tile // CH
    mesh = plsc.VectorSubcoreMesh(core_axis_name="c", subcore_axis_name="s")

    @functools.partial(
        pl.kernel, mesh=mesh,
        compiler_params=pltpu.CompilerParams(use_tc_tiling_on_sc=False),
        out_type=jax.ShapeDtypeStruct((2, NPAD, R), jnp.float32),
        scratch_types=[
            pltpu.VMEM((CH,), jnp.int32),
            pltpu.VMEM((CH,), jnp.int32),
            pltpu.VMEM((CH, R), jnp.float32),
            pltpu.VMEM((CH, 16), jnp.float32),
            pltpu.VMEM_SHARED((NPAD, R), jnp.float32),
            pltpu.SemaphoreType.DMA,
            pltpu.SemaphoreType.DMA,
        ],
    )
    def edge_pass(htab, ttab, src_hbm, dst_hbm, out_hbm,
                  src_v, dst_v, buf, tbuf, accum, sem1, sem2):
        cid = lax.axis_index("c")
        sid = lax.axis_index("s")
        wid = cid * 16 + sid
        row0 = sid * rows_per_tile

        def zrow(r, carry):
            for j in range(R // 16):
                buf[r, pl.ds(j * 16, 16)] = jnp.zeros((16,), jnp.float32)
            return carry
        lax.fori_loop(0, CH, zrow, 0)
        for j in range(nzcopy):
            pltpu.sync_copy(buf, accum.at[pl.ds(row0 + j * CH, CH)])
        plsc.subcore_barrier()

        base = wid * epw

        def chunk(i, carry):
            off = base + i * CH
            pltpu.sync_copy(src_hbm.at[pl.ds(off, CH)], src_v)
            pltpu.sync_copy(dst_hbm.at[pl.ds(off, CH)], dst_v)
            cp1 = pltpu.async_copy(htab.at[src_v], buf, sem1)
            cp2 = pltpu.async_copy(ttab.at[dst_v], tbuf, sem2)
            cp1.wait()
            cp2.wait()

            def edge(e, c2):
                s = buf[e, pl.ds(dh, 16)] + tbuf[e, :]
                s = jnp.maximum(s, s * 0.2)
                w = jnp.exp(s)
                buf[e, pl.ds(dh, 16)] = w
                for hd in range(dh // 16):
                    buf[e, pl.ds(hd * 16, 16)] = buf[e, pl.ds(hd * 16, 16)] * w[hd]
                return c2
            lax.fori_loop(0, CH, edge, 0)
            pltpu.sync_copy(buf, accum.at[dst_v], add=True)
            return carry
        lax.fori_loop(0, NCHUNK, chunk, 0)

        plsc.subcore_barrier()
        pltpu.sync_copy(accum.at[pl.ds(row0, rows_per_tile)],
                        out_hbm.at[cid, pl.ds(row0, rows_per_tile)])

    return edge_pass


_edge_pass_1 = _make_edge_pass(IN_CH)
_edge_pass_2 = _make_edge_pass(HID)


def kernel(x, edge_index, edge_attr, batch, W1, a_src1, a_dst1, b1,
           W2, a_src2, a_dst2, b2):
    del edge_attr
    W1r = W1.reshape(IN_CH, HEADS, HID)
    ws1 = jnp.einsum('ihc,hc->ih', W1r, a_src1)
    wd1 = jnp.einsum('ihc,hc->ih', W1r, a_dst1)
    B1a = jnp.concatenate([W1, ws1, ws1], axis=1)
    B1b = jnp.concatenate([wd1, wd1], axis=1)
    ws2 = (W2 @ a_src2[0])[:, None]
    wd2 = (W2 @ a_dst2[0])[:, None]
    B2a = jnp.concatenate([W2, jnp.tile(ws2, (1, HID))], axis=1)
    B2b = jnp.tile(wd2, (1, HID))

    xp = jnp.zeros((NPAD, IN_CH), jnp.float32).at[:N].set(x)
    idx_dtype = edge_index.dtype
    loop = jnp.arange(N, dtype=idx_dtype)
    fake = jnp.full((EPAD - E - N,), N, dtype=idx_dtype)
    src = jnp.concatenate([edge_index[0], loop, fake]).astype(jnp.int32)
    dst = jnp.concatenate([edge_index[1], loop, fake]).astype(jnp.int32)

    hrow = lax.broadcasted_iota(jnp.int32, (HEADS, IN_CH), 0)
    hcol = lax.broadcasted_iota(jnp.int32, (HEADS, IN_CH), 1) // HID
    e_exp = (hrow == hcol).astype(jnp.float32)

    batch_pad = jnp.concatenate(
        [batch.astype(jnp.int32),
         jnp.full((NPAD - N,), NUM_GRAPHS, jnp.int32)]).reshape(GRID, 1, BLK)

    h1, t1 = _mm2(xp, B1a, B1b)
    p1 = _edge_pass_1(h1, t1, src, dst)
    h2, t2 = _norm_mm2(p1, e_exp, b1.reshape(1, IN_CH), B2a, B2b)
    p2 = _edge_pass_2(h2, t2, src, dst)
    return _pool(p2, batch_pad, b2.reshape(1, HID))
